# baseline (speedup 1.0000x reference)
import functools

import jax
import jax.numpy as jnp
from jax import lax
from jax.experimental import pallas as pl
from jax.experimental.pallas import tpu as pltpu
from jax.experimental.pallas import tpu_sc as plsc

_W = 1024
_NB = 2048


def _make_sc_sort(nrows, n, nc, ns):
    assert n % _W == 0 and nrows <= nc * ns
    nw = n // _W
    total = nrows * n
    mesh = plsc.VectorSubcoreMesh(
        core_axis_name="c", subcore_axis_name="s", num_cores=nc, num_subcores=ns)
    outs = (
        jax.ShapeDtypeStruct((total,), jnp.float32),
        jax.ShapeDtypeStruct((total,), jnp.float32),
        jax.ShapeDtypeStruct((total,), jnp.float32),
    )

    @functools.partial(
        pl.kernel, out_type=outs, mesh=mesh,
        compiler_params=pltpu.CompilerParams(needs_layout_passes=False),
        scratch_types=[
            pltpu.VMEM((_W,), jnp.float32),
            [pltpu.VMEM((128,), jnp.int32)] * (_W // 128),
            pltpu.VMEM((2 * 16 * _NB,), jnp.int32),
            pltpu.VMEM((_NB,), jnp.int32),
            pltpu.VMEM((_W,), jnp.float32),
            pltpu.VMEM((16,), jnp.int32),
            pltpu.SemaphoreType.DMA,
        ])
    def k(x_hbm, out_hbm, ta_hbm, tb_hbm, vbuf, idxb, hist, offs, sbuf, t16a, sem):
        wid = lax.axis_index("s") * nc + lax.axis_index("c")
        iota = lax.broadcasted_iota(jnp.int32, (16,), 0)
        idxm1 = jnp.maximum(iota - 1, 0)
        idxp1 = jnp.minimum(iota + 1, 15)
        fifteen = jnp.full((16,), 15, jnp.int32)
        laneoff = iota * _NB
        ones = jnp.full((16,), 1, jnp.int32)
        zeros16 = jnp.zeros((16,), jnp.int32)

        def key_of(v):
            kb = lax.bitcast_convert_type(v, jnp.int32)
            xm = lax.shift_right_arithmetic(kb, 31) | jnp.int32(-2147483648)
            return kb ^ xm

        @pl.when(wid < nrows)
        def _():
            rowbase = wid * n

            def do_pass(src, dst, shift, dmask):
                def zh(c, _):
                    hist[pl.ds(c * 16, 16)] = zeros16
                    return 0
                lax.fori_loop(0, 2 * 16 * _NB // 16, zh, 0)

                def hw(w, _):
                    pltpu.sync_copy(src.at[pl.ds(rowbase + w * _W, _W)], vbuf)

                    def hv(i, _):
                        v = vbuf[pl.ds(i * 16, 16)]
                        d = lax.shift_right_logical(key_of(v), shift) & dmask
                        bank = (i & 1) * (16 * _NB)
                        plsc.addupdate_scatter(hist, [bank + laneoff + d], ones)
                        return 0
                    lax.fori_loop(0, _W // 16, hv, 0)
                    return 0
                lax.fori_loop(0, nw, hw, 0)

                def pf(c, carry):
                    acc = hist[pl.ds(c * 16, 16)]
                    for l in range(1, 32):
                        acc = acc + hist[pl.ds(l * _NB + c * 16, 16)]
                    inc = plsc.cumsum(acc)
                    offs[pl.ds(c * 16, 16)] = (inc - acc) + carry
                    return carry + jnp.take_along_axis(inc, fifteen, axis=0)
                lax.fori_loop(0, _NB // 16, pf,
                              jnp.full((16,), rowbase, jnp.int32))

                def pw(w, _):
                    pltpu.sync_copy(src.at[pl.ds(rowbase + w * _W, _W)], vbuf)

                    for b in range(_W // 128):
                        def pr(r, _, b=b):
                            i = b * 8 + r
                            v = vbuf[pl.ds(i * 16, 16)]
                            d = lax.shift_right_logical(key_of(v), shift) & dmask
                            sk, sd = plsc.sort_key_val((d << 4) | iota, d)
                            slane = sk & 15
                            prevd = jnp.take_along_axis(sd, idxm1, axis=0)
                            nextd = jnp.take_along_axis(sd, idxp1, axis=0)
                            is_start = (iota == 0) | (sd != prevd)
                            is_end = (iota == 15) | (sd != nextd)
                            startpos = plsc.cummax(
                                jnp.where(is_start, iota, zeros16))
                            rank_s = iota - startpos
                            base_s = plsc.load_gather(offs, [sd])
                            plsc.addupdate_scatter(
                                offs, [sd], rank_s + ones, mask=is_end)
                            vi = lax.bitcast_convert_type(v, jnp.int32)
                            vs = lax.bitcast_convert_type(
                                jnp.take_along_axis(vi, slane, axis=0),
                                jnp.float32)
                            sbuf[pl.ds(i * 16, 16)] = vs
                            idxb[b][pl.ds(r * 16, 16)] = base_s + rank_s
                            return 0
                        lax.fori_loop(0, 8, pr, 0)
                    copies = [
                        pltpu.async_copy(
                            sbuf.at[pl.ds(b * 128, 128)], dst.at[idxb[b]], sem)
                        for b in range(_W // 128)]
                    for cpy in copies:
                        cpy.wait()
                    return 0
                lax.fori_loop(0, nw, pw, 0)

            do_pass(x_hbm, ta_hbm, 0, 2047)
            do_pass(ta_hbm, tb_hbm, 11, 2047)
            do_pass(tb_hbm, out_hbm, 22, 1023)

    return k


def kernel(x):
    B, n = x.shape
    npad = ((n + _W - 1) // _W) * _W
    xp = jnp.pad(x, ((0, 0), (0, npad - n)),
                 constant_values=jnp.float32(jnp.inf))
    info = plsc.get_sparse_core_info()
    k = _make_sc_sort(B, npad, info.num_cores, info.num_subcores)
    out, _, _ = k(xp.reshape(-1))
    return out.reshape(B, npad)[:, :n]


# speedup vs baseline: 11.8120x; 11.8120x over previous
import functools

import jax
import jax.numpy as jnp
from jax import lax
from jax.experimental import pallas as pl


def _rollrows(x, s):
    return jnp.concatenate([x[s:, :], x[:s, :]], axis=0)


def _rolllanes(x, s):
    return jnp.concatenate([x[:, s:], x[:, :s]], axis=1)


def _bitonic_cm_kernel(x_ref, o_ref, *, L, CH):
    N = 1 << L
    R = N // 128
    La = L - 7
    lch = CH.bit_length() - 1
    nchunks = R // CH

    def cp(c, _):
        cb = c * CH
        o_ref[0, pl.ds(cb, CH), :] = x_ref[0, pl.ds(cb, CH), :]
        return 0

    lax.fori_loop(0, nchunks, cp, 0)

    row_iota = lax.broadcasted_iota(jnp.int32, (CH, 1), 0)
    lane_iota = lax.broadcasted_iota(jnp.int32, (1, 128), 1)

    def lanemask(bit):
        return ((lane_iota >> bit) & 1) == 0

    def rowmask(bit):
        return ((row_iota >> bit) & 1) == 0

    def stage_in_regs(x, cb, k, j):
        if j >= La:
            s = 1 << (j - La)
            lob = lanemask(j - La)
            part = jnp.where(lob, _rolllanes(x, s), _rolllanes(x, 128 - s))
            mn = jnp.minimum(x, part)
            mx = jnp.maximum(x, part)
            if k == L:
                tm = lob
            else:
                tm = lob == lanemask(k - La)
            return jnp.where(tm, mn, mx)
        s = 1 << j
        if k < lch:
            lob = rowmask(j)
            part = jnp.where(lob, _rollrows(x, s), _rollrows(x, CH - s))
            mn = jnp.minimum(x, part)
            mx = jnp.maximum(x, part)
            tm = lob == rowmask(k)
            return jnp.where(tm, mn, mx)
        g = CH // (2 * s)
        x4 = x.reshape(g, 2, s, 128)
        lo = x4[:, 0]
        hi = x4[:, 1]
        mn = jnp.minimum(lo, hi)
        mx = jnp.maximum(lo, hi)
        if k == L:
            nlo, nhi = mn, mx
        elif k >= La:
            am = lanemask(k - La)
            nlo = jnp.where(am, mn, mx)
            nhi = jnp.where(am, mx, mn)
        else:
            asc = ((cb >> k) & 1) == 0
            nlo = jnp.where(asc, mn, mx)
            nhi = jnp.where(asc, mx, mn)
        return jnp.concatenate(
            [nlo[:, None], nhi[:, None]], axis=1).reshape(CH, 128)

    def emit_chunk_run(k, js):
        if not js:
            return

        def body(c, _):
            cb = c * CH
            x = o_ref[0, pl.ds(cb, CH), :]
            for j in js:
                x = stage_in_regs(x, cb, k, j)
            o_ref[0, pl.ds(cb, CH), :] = x
            return 0

        lax.fori_loop(0, nchunks, body, 0)

    def emit_slab_stage(k, j):
        s = 1 << j
        ratio = s // CH
        for m in range((R // 2) // CH):
            g, t = divmod(m, ratio)
            lo_base = g * 2 * s + t * CH
            hi_base = lo_base + s
            lo = o_ref[0, pl.ds(lo_base, CH), :]
            hi = o_ref[0, pl.ds(hi_base, CH), :]
            mn = jnp.minimum(lo, hi)
            mx = jnp.maximum(lo, hi)
            if k == L:
                nlo, nhi = mn, mx
            elif k >= La:
                am = lanemask(k - La)
                nlo = jnp.where(am, mn, mx)
                nhi = jnp.where(am, mx, mn)
            else:
                if ((lo_base >> k) & 1) == 0:
                    nlo, nhi = mn, mx
                else:
                    nlo, nhi = mx, mn
            o_ref[0, pl.ds(lo_base, CH), :] = nlo
            o_ref[0, pl.ds(hi_base, CH), :] = nhi

    for k in range(1, L + 1):
        lane_js = [j for j in range(k - 1, -1, -1) if j >= La]
        slab_js = [j for j in range(min(k - 1, La - 1), -1, -1) if j >= lch]
        chunk_js = [j for j in range(min(k - 1, lch - 1), -1, -1)]
        emit_chunk_run(k, lane_js)
        for j in slab_js:
            emit_slab_stage(k, j)
        emit_chunk_run(k, chunk_js)


def _sort_padded_cm(x3, L, CH, interpret=False):
    B, R, _ = x3.shape
    return pl.pallas_call(
        functools.partial(_bitonic_cm_kernel, L=L, CH=CH),
        grid=(B,),
        in_specs=[pl.BlockSpec((1, R, 128), lambda i: (i, 0, 0))],
        out_specs=pl.BlockSpec((1, R, 128), lambda i: (i, 0, 0)),
        out_shape=jax.ShapeDtypeStruct((B, R, 128), jnp.float32),
        interpret=interpret,
    )(x3)


def kernel(x):
    B, n = x.shape
    L = max(8, (n - 1).bit_length())
    N = 1 << L
    R = N // 128
    CH = min(256, R)
    xp = jnp.pad(x, ((0, 0), (0, N - n)), constant_values=jnp.float32(jnp.inf))
    out = _sort_padded_cm(xp.reshape(B, R, 128), L, CH)
    return out.transpose(0, 2, 1).reshape(B, N)[:, :n]


# speedup vs baseline: 12.6336x; 1.0696x over previous
import functools

import jax
import jax.numpy as jnp
from jax import lax
from jax.experimental import pallas as pl


def _rollrows(x, s):
    return jnp.concatenate([x[s:, :], x[:s, :]], axis=0)


def _rolllanes(x, s):
    return jnp.concatenate([x[:, s:], x[:, :s]], axis=1)


def _bitonic_cm_kernel(x_ref, o_ref, *, L, CH):
    N = 1 << L
    R = N // 128
    La = L - 7
    lch = CH.bit_length() - 1
    nchunks = R // CH

    def cp(c, _):
        cb = c * CH
        o_ref[0, pl.ds(cb, CH), :] = x_ref[0, pl.ds(cb, CH), :]
        return 0

    lax.fori_loop(0, nchunks, cp, 0)

    row_iota = lax.broadcasted_iota(jnp.int32, (CH, 1), 0)
    lane_iota = lax.broadcasted_iota(jnp.int32, (1, 128), 1)

    def lanemask(bit):
        return ((lane_iota >> bit) & 1) == 0

    def rowmask(bit):
        return ((row_iota >> bit) & 1) == 0

    def stage_in_regs(x, cb, k, j):
        if j >= La:
            s = 1 << (j - La)
            lob = lanemask(j - La)
            part = jnp.where(lob, _rolllanes(x, s), _rolllanes(x, 128 - s))
            mn = jnp.minimum(x, part)
            mx = jnp.maximum(x, part)
            if k == L:
                tm = lob
            else:
                tm = lob == lanemask(k - La)
            return jnp.where(tm, mn, mx)
        s = 1 << j
        if k < lch:
            lob = rowmask(j)
            part = jnp.where(lob, _rollrows(x, s), _rollrows(x, CH - s))
            mn = jnp.minimum(x, part)
            mx = jnp.maximum(x, part)
            tm = lob == rowmask(k)
            return jnp.where(tm, mn, mx)
        g = CH // (2 * s)
        x4 = x.reshape(g, 2, s, 128)
        lo = x4[:, 0]
        hi = x4[:, 1]
        mn = jnp.minimum(lo, hi)
        mx = jnp.maximum(lo, hi)
        if k == L:
            nlo, nhi = mn, mx
        elif k >= La:
            am = lanemask(k - La)
            nlo = jnp.where(am, mn, mx)
            nhi = jnp.where(am, mx, mn)
        else:
            asc = ((cb >> k) & 1) == 0
            nlo = jnp.where(asc, mn, mx)
            nhi = jnp.where(asc, mx, mn)
        return jnp.concatenate(
            [nlo[:, None], nhi[:, None]], axis=1).reshape(CH, 128)

    def emit_chunk_run(k, js):
        if not js:
            return

        def body(c, _):
            cb = c * CH
            x = o_ref[0, pl.ds(cb, CH), :]
            for j in js:
                x = stage_in_regs(x, cb, k, j)
            o_ref[0, pl.ds(cb, CH), :] = x
            return 0

        lax.fori_loop(0, nchunks, body, 0)

    def emit_slab_stage(k, j):
        s = 1 << j
        ratio = s // CH
        for m in range((R // 2) // CH):
            g, t = divmod(m, ratio)
            lo_base = g * 2 * s + t * CH
            hi_base = lo_base + s
            lo = o_ref[0, pl.ds(lo_base, CH), :]
            hi = o_ref[0, pl.ds(hi_base, CH), :]
            mn = jnp.minimum(lo, hi)
            mx = jnp.maximum(lo, hi)
            if k == L:
                nlo, nhi = mn, mx
            elif k >= La:
                am = lanemask(k - La)
                nlo = jnp.where(am, mn, mx)
                nhi = jnp.where(am, mx, mn)
            else:
                if ((lo_base >> k) & 1) == 0:
                    nlo, nhi = mn, mx
                else:
                    nlo, nhi = mx, mn
            o_ref[0, pl.ds(lo_base, CH), :] = nlo
            o_ref[0, pl.ds(hi_base, CH), :] = nhi

    for k in range(1, L + 1):
        lane_js = [j for j in range(k - 1, -1, -1) if j >= La]
        slab_js = [j for j in range(min(k - 1, La - 1), -1, -1) if j >= lch]
        chunk_js = [j for j in range(min(k - 1, lch - 1), -1, -1)]
        emit_chunk_run(k, lane_js)
        for j in slab_js:
            emit_slab_stage(k, j)
        emit_chunk_run(k, chunk_js)


def _sort_padded_cm(x3, L, CH, interpret=False):
    B, R, _ = x3.shape
    return pl.pallas_call(
        functools.partial(_bitonic_cm_kernel, L=L, CH=CH),
        grid=(B,),
        in_specs=[pl.BlockSpec((1, R, 128), lambda i: (i, 0, 0))],
        out_specs=pl.BlockSpec((1, R, 128), lambda i: (i, 0, 0)),
        out_shape=jax.ShapeDtypeStruct((B, R, 128), jnp.float32),
        interpret=interpret,
    )(x3)


def kernel(x):
    B, n = x.shape
    L = max(8, (n - 1).bit_length())
    N = 1 << L
    R = N // 128
    CH = min(512, R)
    xp = jnp.pad(x, ((0, 0), (0, N - n)), constant_values=jnp.float32(jnp.inf))
    out = _sort_padded_cm(xp.reshape(B, R, 128), L, CH)
    return out.transpose(0, 2, 1).reshape(B, N)[:, :n]
